# speedup vs baseline: 4.1027x; 4.1027x over previous
import functools

import jax
import jax.numpy as jnp
from jax import lax
from jax.experimental import pallas as pl
from jax.experimental.pallas import tpu as pltpu
from jax.experimental.pallas import tpu_sc as plsc

N_NODES = 10000
N_EDGES = 320000
D = 128

NC = 2
NS = 16
NW = NC * NS

CHUNK = 128
CHUNKS = 79
E_PAD = NW * CHUNKS * CHUNK
N_PAD = 10240
DUMMY_DST = N_NODES
ROWS_PER_TILE = N_PAD // NS


def _segment_sum_sc(table, src3, dst3):
    mesh = plsc.VectorSubcoreMesh(
        core_axis_name="c", subcore_axis_name="s", num_cores=NC, num_subcores=NS
    )

    @functools.partial(
        pl.kernel,
        mesh=mesh,
        out_type=jax.ShapeDtypeStruct((NC, N_PAD, D), jnp.float32),
        scratch_types=[
            pltpu.VMEM((CHUNKS, CHUNK), jnp.int32),
            pltpu.VMEM((CHUNKS, CHUNK), jnp.int32),
            pltpu.VMEM((CHUNK, D), jnp.float32),
            pltpu.VMEM_SHARED((N_PAD, D), jnp.float32),
            pltpu.SemaphoreType.DMA,
        ],
    )
    def k(table_hbm, src_hbm, dst_hbm, out_hbm, sidx, didx, rows, acc, sem):
        c = lax.axis_index("c")
        s = lax.axis_index("s")
        wid = s * NC + c

        def zero_body(i, carry):
            r = i // 8
            col = (i % 8) * 16
            rows[r, pl.ds(col, 16)] = jnp.zeros((16,), jnp.float32)
            return carry

        lax.fori_loop(0, CHUNK * 8, zero_body, 0)
        for t in range(ROWS_PER_TILE // CHUNK):
            pltpu.sync_copy(rows, acc.at[pl.ds(s * ROWS_PER_TILE + t * CHUNK, CHUNK)])
        plsc.subcore_barrier()

        pltpu.sync_copy(src_hbm.at[wid], sidx)
        pltpu.sync_copy(dst_hbm.at[wid], didx)

        def body(j, carry):
            pltpu.async_copy(table_hbm.at[sidx.at[j]], rows, sem).wait()
            pltpu.sync_copy(rows, acc.at[didx.at[j]], add=True)
            return carry

        lax.fori_loop(0, CHUNKS, body, 0)
        plsc.subcore_barrier()

        pltpu.sync_copy(
            acc.at[pl.ds(s * ROWS_PER_TILE, ROWS_PER_TILE)],
            out_hbm.at[c, pl.ds(s * ROWS_PER_TILE, ROWS_PER_TILE)],
        )

    return k(table, src3, dst3)


_BLK = 1280


def _combine_linear_relu(p, W, brow):
    def body(p_ref, w_ref, b_ref, o_ref):
        ssum = p_ref[0] + p_ref[1]
        o_ref[...] = jnp.maximum(
            jnp.dot(ssum, w_ref[...], preferred_element_type=jnp.float32)
            + b_ref[...],
            0.0,
        )

    return pl.pallas_call(
        body,
        grid=(N_PAD // _BLK,),
        in_specs=[
            pl.BlockSpec((2, _BLK, D), lambda i: (0, i, 0)),
            pl.BlockSpec((D, D), lambda i: (0, 0)),
            pl.BlockSpec((1, D), lambda i: (0, 0)),
        ],
        out_specs=pl.BlockSpec((_BLK, D), lambda i: (i, 0)),
        out_shape=jax.ShapeDtypeStruct((N_PAD, D), jnp.float32),
    )(p, W, brow)


def _combine_linear_relu_linear(q, W2, b2row, Wab, babrow):
    def body(q_ref, w2_ref, b2_ref, wab_ref, bab_ref, o_ref):
        ssum = q_ref[0] + q_ref[1]
        h = jnp.maximum(
            jnp.dot(ssum, w2_ref[...], preferred_element_type=jnp.float32)
            + b2_ref[...],
            0.0,
        )
        o_ref[...] = (
            jnp.dot(h, wab_ref[...], preferred_element_type=jnp.float32)
            + bab_ref[...]
        )

    return pl.pallas_call(
        body,
        grid=(N_PAD // _BLK,),
        in_specs=[
            pl.BlockSpec((2, _BLK, D), lambda i: (0, i, 0)),
            pl.BlockSpec((D, D), lambda i: (0, 0)),
            pl.BlockSpec((1, D), lambda i: (0, 0)),
            pl.BlockSpec((D, D), lambda i: (0, 0)),
            pl.BlockSpec((1, D), lambda i: (0, 0)),
        ],
        out_specs=pl.BlockSpec((_BLK, D), lambda i: (i, 0)),
        out_shape=jax.ShapeDtypeStruct((N_PAD, D), jnp.float32),
    )(q, W2, b2row, Wab, babrow)


@jax.jit
def kernel(x, edge_index, W1, b1, W2, b2, Wa, ba, Wb, bb):
    src = edge_index[0].astype(jnp.int32)
    dst = edge_index[1].astype(jnp.int32)
    pad = E_PAD - N_EDGES
    src3 = jnp.concatenate([src, jnp.zeros((pad,), jnp.int32)]).reshape(
        NW, CHUNKS, CHUNK
    )
    dst3 = jnp.concatenate(
        [dst, jnp.full((pad,), DUMMY_DST, jnp.int32)]
    ).reshape(NW, CHUNKS, CHUNK)

    d_a, d_b = Wa.shape[1], Wb.shape[1]
    Wab = jnp.zeros((D, D), jnp.float32)
    Wab = Wab.at[:, :d_a].set(Wa).at[:, d_a : d_a + d_b].set(Wb)
    bab = jnp.zeros((1, D), jnp.float32)
    bab = bab.at[0, :d_a].set(ba).at[0, d_a : d_a + d_b].set(bb)

    p1 = _segment_sum_sc(x, src3, dst3)
    h1 = _combine_linear_relu(p1, W1, b1.reshape(1, D))
    p2 = _segment_sum_sc(h1, src3, dst3)
    xab = _combine_linear_relu_linear(p2, W2, b2.reshape(1, D), Wab, bab)

    xa = xab[:N_NODES, :d_a]
    xb = xab[:N_NODES, d_a : d_a + d_b]
    return (xa, xb)
